# Initial kernel scaffold; baseline (speedup 1.0000x reference)
import functools

import jax
import jax.numpy as jnp
from jax import lax
from jax.experimental import pallas as pl
from jax.experimental.pallas import tpu as pltpu
from jax.experimental.pallas import tpu_sc as plsc

N = 10000
E = 320000
IN_F = 128
HID = 128
NCLS = 64

NC = 2
NS = 16
NW = NC * NS
EPW = E // NW
K = 80
ITERS = EPW // K
N_PAD = 10240
NPT = N_PAD // NS
RPT = N // NS

_mesh = plsc.VectorSubcoreMesh(core_axis_name="c", subcore_axis_name="s")


@functools.partial(
    pl.kernel,
    out_type=jax.ShapeDtypeStruct((2 * 2 * N_PAD,), jnp.float32),
    mesh=_mesh,
    scratch_types=[
        pltpu.VMEM((K,), jnp.int32),
        pltpu.VMEM((K,), jnp.int32),
        pltpu.VMEM((K,), jnp.float32),
        pltpu.VMEM_SHARED((N_PAD,), jnp.float32),
        pltpu.VMEM_SHARED((N_PAD,), jnp.float32),
    ],
)
def _sc_degrees(src_hbm, dst_hbm, zeros_hbm, out_hbm,
                src_v, dst_v, ones_v, dego_sh, degi_sh):
    cid = lax.axis_index("c")
    sid = lax.axis_index("s")
    pltpu.sync_copy(zeros_hbm.at[pl.ds(sid * NPT, NPT)],
                    dego_sh.at[pl.ds(sid * NPT, NPT)])
    pltpu.sync_copy(zeros_hbm.at[pl.ds(sid * NPT, NPT)],
                    degi_sh.at[pl.ds(sid * NPT, NPT)])
    for j in range(K // 16):
        ones_v[pl.ds(j * 16, 16)] = jnp.ones((16,), jnp.float32)
    plsc.subcore_barrier()

    base = (cid * NS + sid) * EPW

    def body(i, carry):
        off = base + i * K
        pltpu.sync_copy(src_hbm.at[pl.ds(off, K)], src_v)
        pltpu.sync_copy(dst_hbm.at[pl.ds(off, K)], dst_v)
        pltpu.sync_copy(ones_v, dego_sh.at[src_v], add=True)
        pltpu.sync_copy(ones_v, degi_sh.at[dst_v], add=True)
        return carry

    lax.fori_loop(0, ITERS, body, 0)
    plsc.subcore_barrier()
    pltpu.sync_copy(dego_sh.at[pl.ds(sid * NPT, NPT)],
                    out_hbm.at[pl.ds(cid * 2 * N_PAD + sid * NPT, NPT)])
    pltpu.sync_copy(degi_sh.at[pl.ds(sid * NPT, NPT)],
                    out_hbm.at[pl.ds(cid * 2 * N_PAD + N_PAD + sid * NPT, NPT)])


def _make_sc_segsum(D):
    @functools.partial(
        pl.kernel,
        out_type=jax.ShapeDtypeStruct((2 * N, D), jnp.float32),
        mesh=_mesh,
        scratch_types=[
            pltpu.VMEM((K,), jnp.int32),
            pltpu.VMEM((K,), jnp.int32),
            pltpu.VMEM((K, D), jnp.float32),
            pltpu.VMEM_SHARED((N, D), jnp.float32),
            pltpu.SemaphoreType.DMA,
        ],
    )
    def segsum(h_hbm, src_hbm, dst_hbm, zeros_hbm, out_hbm,
               src_v, dst_v, rows_v, acc_sh, sem):
        cid = lax.axis_index("c")
        sid = lax.axis_index("s")
        pltpu.sync_copy(zeros_hbm.at[pl.ds(sid * RPT, RPT)],
                        acc_sh.at[pl.ds(sid * RPT, RPT)])
        plsc.subcore_barrier()

        base = (cid * NS + sid) * EPW

        def body(i, carry):
            off = base + i * K
            pltpu.sync_copy(src_hbm.at[pl.ds(off, K)], src_v)
            pltpu.sync_copy(dst_hbm.at[pl.ds(off, K)], dst_v)
            pltpu.async_copy(h_hbm.at[src_v], rows_v, sem).wait()
            pltpu.sync_copy(rows_v, acc_sh.at[dst_v], add=True)
            return carry

        lax.fori_loop(0, ITERS, body, 0)
        plsc.subcore_barrier()
        pltpu.sync_copy(acc_sh.at[pl.ds(sid * RPT, RPT)],
                        out_hbm.at[pl.ds(cid * N + sid * RPT, RPT)])

    return segsum


_sc_segsum_hid = _make_sc_segsum(HID)
_sc_segsum_cls = _make_sc_segsum(NCLS)


_BM = 1000
_GRID = N // _BM


def _mm_body(x_ref, w_ref, o_ref):
    o_ref[...] = jnp.dot(x_ref[...], w_ref[...],
                         preferred_element_type=jnp.float32)


def _tc_matmul(x, w):
    d_in, d_out = w.shape
    return pl.pallas_call(
        _mm_body,
        grid=(_GRID,),
        in_specs=[
            pl.BlockSpec((_BM, d_in), lambda i: (i, 0)),
            pl.BlockSpec((d_in, d_out), lambda i: (0, 0)),
        ],
        out_specs=pl.BlockSpec((_BM, d_out), lambda i: (i, 0)),
        out_shape=jax.ShapeDtypeStruct((N, d_out), jnp.float32),
    )(x, w)


def _scale_body(z_ref, deg_ref, h_ref, no_ref, ni_ref):
    d = deg_ref[...]
    do = d[0, 0] + d[1, 0]
    di = d[0, 1] + d[1, 1]
    no = lax.rsqrt(jnp.maximum(do, 1.0))
    ni = lax.rsqrt(jnp.maximum(di, 1.0))
    no_ref[...] = no
    ni_ref[...] = ni
    h_ref[...] = z_ref[...] * no


def _tc_scale(z1, degs):
    return pl.pallas_call(
        _scale_body,
        grid=(_GRID,),
        in_specs=[
            pl.BlockSpec((_BM, HID), lambda i: (i, 0)),
            pl.BlockSpec((2, 2, _BM, 1), lambda i: (0, 0, i, 0)),
        ],
        out_specs=[
            pl.BlockSpec((_BM, HID), lambda i: (i, 0)),
            pl.BlockSpec((_BM, 1), lambda i: (i, 0)),
            pl.BlockSpec((_BM, 1), lambda i: (i, 0)),
        ],
        out_shape=[
            jax.ShapeDtypeStruct((N, HID), jnp.float32),
            jax.ShapeDtypeStruct((N, 1), jnp.float32),
            jax.ShapeDtypeStruct((N, 1), jnp.float32),
        ],
    )(z1, degs)


def _layer2_body(p0_ref, p1_ref, ni_ref, no_ref, b1_ref, w2_ref,
                 x1_ref, h2_ref):
    x1 = (p0_ref[...] + p1_ref[...]) * ni_ref[...] + b1_ref[...]
    x1_ref[...] = x1
    x = jnp.maximum(x1, 0.0)
    h2_ref[...] = jnp.dot(x, w2_ref[...],
                          preferred_element_type=jnp.float32) * no_ref[...]


def _tc_layer2(p0, p1, ni, no, b1, w2):
    return pl.pallas_call(
        _layer2_body,
        grid=(_GRID,),
        in_specs=[
            pl.BlockSpec((_BM, HID), lambda i: (i, 0)),
            pl.BlockSpec((_BM, HID), lambda i: (i, 0)),
            pl.BlockSpec((_BM, 1), lambda i: (i, 0)),
            pl.BlockSpec((_BM, 1), lambda i: (i, 0)),
            pl.BlockSpec((1, HID), lambda i: (0, 0)),
            pl.BlockSpec((HID, NCLS), lambda i: (0, 0)),
        ],
        out_specs=[
            pl.BlockSpec((_BM, HID), lambda i: (i, 0)),
            pl.BlockSpec((_BM, NCLS), lambda i: (i, 0)),
        ],
        out_shape=[
            jax.ShapeDtypeStruct((N, HID), jnp.float32),
            jax.ShapeDtypeStruct((N, NCLS), jnp.float32),
        ],
    )(p0, p1, ni, no, b1, w2)


def _final_body(q0_ref, q1_ref, ni_ref, b2_ref, o_ref):
    o_ref[...] = (q0_ref[...] + q1_ref[...]) * ni_ref[...] + b2_ref[...]


def _tc_final(q0, q1, ni, b2):
    return pl.pallas_call(
        _final_body,
        grid=(_GRID,),
        in_specs=[
            pl.BlockSpec((_BM, NCLS), lambda i: (i, 0)),
            pl.BlockSpec((_BM, NCLS), lambda i: (i, 0)),
            pl.BlockSpec((_BM, 1), lambda i: (i, 0)),
            pl.BlockSpec((1, NCLS), lambda i: (0, 0)),
        ],
        out_specs=pl.BlockSpec((_BM, NCLS), lambda i: (i, 0)),
        out_shape=jax.ShapeDtypeStruct((N, NCLS), jnp.float32),
    )(q0, q1, ni, b2)


def kernel(features, edge_index, W1, b1, W2, b2):
    src = edge_index[0]
    dst = edge_index[1]

    zeros_1d = jnp.zeros((N_PAD,), jnp.float32)
    zeros_hid = jnp.zeros((N, HID), jnp.float32)
    zeros_cls = jnp.zeros((N, NCLS), jnp.float32)

    deg_flat = _sc_degrees(src, dst, zeros_1d)
    degs = deg_flat.reshape(2, 2, N_PAD)[:, :, :N].reshape(2, 2, N, 1)

    z1 = _tc_matmul(features, W1)
    h1, no, ni = _tc_scale(z1, degs)

    p = _sc_segsum_hid(h1, src, dst, zeros_hid).reshape(2, N, HID)
    x1, h2 = _tc_layer2(p[0], p[1], ni, no, b1.reshape(1, HID), W2)

    q = _sc_segsum_cls(h2, src, dst, zeros_cls).reshape(2, N, NCLS)
    x2 = _tc_final(q[0], q[1], ni, b2.reshape(1, NCLS))

    return (x2, x1)


# speedup vs baseline: 4.5373x; 4.5373x over previous
import functools

import jax
import jax.numpy as jnp
from jax import lax
from jax.experimental import pallas as pl
from jax.experimental.pallas import tpu as pltpu
from jax.experimental.pallas import tpu_sc as plsc

N = 10000
E = 320000
IN_F = 128
HID = 128
NCLS = 64

NC = 2
NS = 16
NW = NC * NS
EPW = E // NW
K = 80
ITERS = EPW // K
N_PAD = 10240
NPT = N_PAD // NS
RPT = N_PAD // NS

_mesh = plsc.VectorSubcoreMesh(core_axis_name="c", subcore_axis_name="s")


@functools.partial(
    pl.kernel,
    out_type=jax.ShapeDtypeStruct((2 * 2 * N_PAD,), jnp.float32),
    mesh=_mesh,
    scratch_types=[
        pltpu.VMEM((K,), jnp.int32),
        pltpu.VMEM((K,), jnp.int32),
        pltpu.VMEM((K,), jnp.float32),
        pltpu.VMEM_SHARED((N_PAD,), jnp.float32),
        pltpu.VMEM_SHARED((N_PAD,), jnp.float32),
    ],
)
def _sc_degrees(src_hbm, dst_hbm, zeros_hbm, out_hbm,
                src_v, dst_v, ones_v, dego_sh, degi_sh):
    cid = lax.axis_index("c")
    sid = lax.axis_index("s")
    pltpu.sync_copy(zeros_hbm.at[pl.ds(sid * NPT, NPT)],
                    dego_sh.at[pl.ds(sid * NPT, NPT)])
    pltpu.sync_copy(zeros_hbm.at[pl.ds(sid * NPT, NPT)],
                    degi_sh.at[pl.ds(sid * NPT, NPT)])
    for j in range(K // 16):
        ones_v[pl.ds(j * 16, 16)] = jnp.ones((16,), jnp.float32)
    plsc.subcore_barrier()

    base = (cid * NS + sid) * EPW

    def body(i, carry):
        off = base + i * K
        pltpu.sync_copy(src_hbm.at[pl.ds(off, K)], src_v)
        pltpu.sync_copy(dst_hbm.at[pl.ds(off, K)], dst_v)
        pltpu.sync_copy(ones_v, dego_sh.at[src_v], add=True)
        pltpu.sync_copy(ones_v, degi_sh.at[dst_v], add=True)
        return carry

    lax.fori_loop(0, ITERS, body, 0)
    plsc.subcore_barrier()
    pltpu.sync_copy(dego_sh.at[pl.ds(sid * NPT, NPT)],
                    out_hbm.at[pl.ds(cid * 2 * N_PAD + sid * NPT, NPT)])
    pltpu.sync_copy(degi_sh.at[pl.ds(sid * NPT, NPT)],
                    out_hbm.at[pl.ds(cid * 2 * N_PAD + N_PAD + sid * NPT, NPT)])


def _make_sc_segsum(D):
    @functools.partial(
        pl.kernel,
        out_type=jax.ShapeDtypeStruct((2 * N_PAD, D), jnp.float32),
        mesh=_mesh,
        scratch_types=[
            pltpu.VMEM((K,), jnp.int32),
            pltpu.VMEM((K,), jnp.int32),
            pltpu.VMEM((K, D), jnp.float32),
            pltpu.VMEM_SHARED((N_PAD, D), jnp.float32),
            pltpu.SemaphoreType.DMA,
        ],
    )
    def segsum(h_hbm, src_hbm, dst_hbm, zeros_hbm, out_hbm,
               src_v, dst_v, rows_v, acc_sh, sem):
        cid = lax.axis_index("c")
        sid = lax.axis_index("s")
        pltpu.sync_copy(zeros_hbm.at[pl.ds(sid * RPT, RPT)],
                        acc_sh.at[pl.ds(sid * RPT, RPT)])
        plsc.subcore_barrier()

        base = (cid * NS + sid) * EPW

        def body(i, carry):
            off = base + i * K
            pltpu.sync_copy(src_hbm.at[pl.ds(off, K)], src_v)
            pltpu.sync_copy(dst_hbm.at[pl.ds(off, K)], dst_v)
            pltpu.async_copy(h_hbm.at[src_v], rows_v, sem).wait()
            pltpu.sync_copy(rows_v, acc_sh.at[dst_v], add=True)
            return carry

        lax.fori_loop(0, ITERS, body, 0)
        plsc.subcore_barrier()
        pltpu.sync_copy(acc_sh.at[pl.ds(sid * RPT, RPT)],
                        out_hbm.at[pl.ds(cid * N_PAD + sid * RPT, RPT)])

    return segsum


_sc_segsum = _make_sc_segsum(HID)


_BM = 1000
_GRID = N // _BM


def _mm_body(x_ref, w_ref, o_ref):
    o_ref[...] = jnp.dot(x_ref[...], w_ref[...],
                         preferred_element_type=jnp.float32)


def _tc_matmul(x, w):
    d_in, d_out = w.shape
    return pl.pallas_call(
        _mm_body,
        grid=(_GRID,),
        in_specs=[
            pl.BlockSpec((_BM, d_in), lambda i: (i, 0)),
            pl.BlockSpec((d_in, d_out), lambda i: (0, 0)),
        ],
        out_specs=pl.BlockSpec((_BM, d_out), lambda i: (i, 0)),
        out_shape=jax.ShapeDtypeStruct((N, d_out), jnp.float32),
    )(x, w)


def _scale_body(z_ref, deg_ref, h_ref, no_ref, ni_ref):
    d = deg_ref[...]
    do = d[0, 0] + d[1, 0]
    di = d[0, 1] + d[1, 1]
    no = lax.rsqrt(jnp.maximum(do, 1.0))
    ni = lax.rsqrt(jnp.maximum(di, 1.0))
    no_ref[...] = no
    ni_ref[...] = ni
    h_ref[...] = z_ref[...] * no


def _tc_scale(z1, degs):
    return pl.pallas_call(
        _scale_body,
        grid=(_GRID,),
        in_specs=[
            pl.BlockSpec((_BM, HID), lambda i: (i, 0)),
            pl.BlockSpec((2, 2, _BM, 1), lambda i: (0, 0, i, 0)),
        ],
        out_specs=[
            pl.BlockSpec((_BM, HID), lambda i: (i, 0)),
            pl.BlockSpec((_BM, 1), lambda i: (i, 0)),
            pl.BlockSpec((_BM, 1), lambda i: (i, 0)),
        ],
        out_shape=[
            jax.ShapeDtypeStruct((N, HID), jnp.float32),
            jax.ShapeDtypeStruct((N, 1), jnp.float32),
            jax.ShapeDtypeStruct((N, 1), jnp.float32),
        ],
    )(z1, degs)


def _layer2_body(p0_ref, p1_ref, ni_ref, no_ref, b1_ref, w2_ref,
                 x1_ref, h2_ref):
    x1 = (p0_ref[...] + p1_ref[...]) * ni_ref[...] + b1_ref[...]
    x1_ref[...] = x1
    x = jnp.maximum(x1, 0.0)
    h2 = jnp.dot(x, w2_ref[...],
                 preferred_element_type=jnp.float32) * no_ref[...]
    h2_ref[...] = jnp.concatenate(
        [h2, jnp.zeros((h2.shape[0], HID - NCLS), jnp.float32)], axis=1)


def _tc_layer2(p0, p1, ni, no, b1, w2):
    return pl.pallas_call(
        _layer2_body,
        grid=(_GRID,),
        in_specs=[
            pl.BlockSpec((_BM, HID), lambda i: (i, 0)),
            pl.BlockSpec((_BM, HID), lambda i: (i, 0)),
            pl.BlockSpec((_BM, 1), lambda i: (i, 0)),
            pl.BlockSpec((_BM, 1), lambda i: (i, 0)),
            pl.BlockSpec((1, HID), lambda i: (0, 0)),
            pl.BlockSpec((HID, NCLS), lambda i: (0, 0)),
        ],
        out_specs=[
            pl.BlockSpec((_BM, HID), lambda i: (i, 0)),
            pl.BlockSpec((_BM, HID), lambda i: (i, 0)),
        ],
        out_shape=[
            jax.ShapeDtypeStruct((N, HID), jnp.float32),
            jax.ShapeDtypeStruct((N, HID), jnp.float32),
        ],
    )(p0, p1, ni, no, b1, w2)


def _final_body(q0_ref, q1_ref, ni_ref, b2_ref, o_ref):
    q = q0_ref[...] + q1_ref[...]
    o_ref[...] = q[:, :NCLS] * ni_ref[...] + b2_ref[...]


def _tc_final(q0, q1, ni, b2):
    return pl.pallas_call(
        _final_body,
        grid=(_GRID,),
        in_specs=[
            pl.BlockSpec((_BM, HID), lambda i: (i, 0)),
            pl.BlockSpec((_BM, HID), lambda i: (i, 0)),
            pl.BlockSpec((_BM, 1), lambda i: (i, 0)),
            pl.BlockSpec((1, NCLS), lambda i: (0, 0)),
        ],
        out_specs=pl.BlockSpec((_BM, NCLS), lambda i: (i, 0)),
        out_shape=jax.ShapeDtypeStruct((N, NCLS), jnp.float32),
    )(q0, q1, ni, b2)


def kernel(features, edge_index, W1, b1, W2, b2):
    src = edge_index[0]
    dst = edge_index[1]

    zeros_1d = jnp.zeros((N_PAD,), jnp.float32)
    zeros_hid = jnp.zeros((N_PAD, HID), jnp.float32)

    deg_flat = _sc_degrees(src, dst, zeros_1d)
    degs = deg_flat.reshape(2, 2, N_PAD)[:, :, :N].reshape(2, 2, N, 1)

    z1 = _tc_matmul(features, W1)
    h1, no, ni = _tc_scale(z1, degs)

    p = _sc_segsum(h1, src, dst, zeros_hid).reshape(2, N_PAD, HID)[:, :N]
    x1, h2 = _tc_layer2(p[0], p[1], ni, no, b1.reshape(1, HID), W2)

    q = _sc_segsum(h2, src, dst, zeros_hid).reshape(2, N_PAD, HID)[:, :N]
    x2 = _tc_final(q[0], q[1], ni, b2.reshape(1, NCLS))

    return (x2, x1)
